# baseline (speedup 1.0000x reference)
import functools

import jax
import jax.numpy as jnp
from jax import lax
from jax.experimental import pallas as pl
from jax.experimental.pallas import tpu as pltpu
from jax.experimental.pallas import tpu_sc as plsc

B, N, IN_DIM, HID, CODE_DIM, N_CODES = 16, 4096, 16, 64, 32, 1024
ROWS = B * N
BLK = 1024
CCHUNK = 128
NCHUNK = N_CODES // CCHUNK
NSLICE = 2

_EPS = 1e-5
_BIG = 3.0e38


def _bf16_dot(a, b):
    return jnp.dot(a.astype(jnp.bfloat16), b.astype(jnp.bfloat16),
                   preferred_element_type=jnp.float32)


def _tc_body(x_ref, w_in_ref, b_in_ref, w1_ref, b1_ref, g1_ref, be1_ref,
             w2_ref, b2_ref, g2_ref, be2_ref, w_out_ref, b_out_ref, cbm2_ref,
             csq_ref, z_ref, tok_ref):
    h = _bf16_dot(x_ref[...], w_in_ref[...]) + b_in_ref[...]
    for w, b, g, be in ((w1_ref, b1_ref, g1_ref, be1_ref),
                        (w2_ref, b2_ref, g2_ref, be2_ref)):
        t = _bf16_dot(h, w[...]) + b[...]
        m = jnp.mean(t, axis=-1, keepdims=True)
        v = jnp.mean((t - m) * (t - m), axis=-1, keepdims=True)
        t = (t - m) / jnp.sqrt(v + _EPS) * g[...] + be[...]
        h = jnp.maximum(t, 0.0) + h
    z = _bf16_dot(h, w_out_ref[...]) + b_out_ref[...]
    z_ref[...] = z

    zb = z.astype(jnp.bfloat16)

    gmin = jnp.full((BLK,), _BIG, jnp.float32)
    gidx = jnp.zeros((BLK,), jnp.int32)
    gchunk = jnp.zeros((BLK,), jnp.int32)
    for c in range(NCHUNK):
        cbm2 = cbm2_ref[c * CCHUNK:(c + 1) * CCHUNK, :]
        mm = lax.dot_general(cbm2, zb, (((1,), (1,)), ((), ())),
                             preferred_element_type=jnp.float32)
        d = mm + csq_ref[c * CCHUNK:(c + 1) * CCHUNK, 0:1]
        cmin = jnp.min(d, axis=0)
        cidx = jnp.argmin(d, axis=0).astype(jnp.int32)
        better = cmin < gmin
        gmin = jnp.where(better, cmin, gmin)
        gidx = jnp.where(better, cidx, gidx)
        gchunk = jnp.where(better, c, gchunk)
    tok_ref[...] = gchunk * CCHUNK + gidx


def _encode_and_tokenize(rows, x2d, w_in, b_in, w1, b1, g1, be1, w2, b2, g2,
                         be2, w_out, b_out, cbm2, csq, interpret=False):
    full = lambda shape: pl.BlockSpec(shape, lambda i: (0,) * len(shape))
    return pl.pallas_call(
        _tc_body,
        grid=(rows // BLK,),
        in_specs=[
            pl.BlockSpec((BLK, IN_DIM), lambda i: (i, 0)),
            full((IN_DIM, HID)), full((1, HID)),
            full((HID, HID)), full((1, HID)), full((1, HID)), full((1, HID)),
            full((HID, HID)), full((1, HID)), full((1, HID)), full((1, HID)),
            full((HID, CODE_DIM)), full((1, CODE_DIM)),
            full((N_CODES, CODE_DIM)),
            full((N_CODES, 1)),
        ],
        out_specs=[
            pl.BlockSpec((BLK, CODE_DIM), lambda i: (i, 0)),
            pl.BlockSpec((BLK,), lambda i: (i,)),
        ],
        out_shape=[
            jax.ShapeDtypeStruct((rows, CODE_DIM), jnp.float32),
            jax.ShapeDtypeStruct((rows,), jnp.int32),
        ],
        interpret=interpret,
    )(x2d, w_in, b_in, w1, b1, g1, be1, w2, b2, g2, be2, w_out, b_out,
      cbm2, csq)


def _sc_gather(rows, codebook, tokens):
    info = plsc.get_sparse_core_info()
    nw = info.num_cores * info.num_subcores
    b_per_w = rows // nw
    chunk = 128
    n_chunks = b_per_w // chunk
    mesh = plsc.VectorSubcoreMesh(core_axis_name="c", subcore_axis_name="s")

    @functools.partial(
        pl.kernel, mesh=mesh,
        out_type=jax.ShapeDtypeStruct((rows, CODE_DIM), jnp.float32),
        scratch_types=[
            pltpu.VMEM((b_per_w,), jnp.int32),
            pltpu.VMEM((b_per_w, CODE_DIM), jnp.float32),
            pltpu.SemaphoreType.DMA,
        ],
        compiler_params=pltpu.CompilerParams(use_tc_tiling_on_sc=False),
    )
    def k(cb_hbm, tok_hbm, out_hbm, idx_v, rows_v, sem):
        wid = lax.axis_index("s") * info.num_cores + lax.axis_index("c")
        base = wid * b_per_w
        pltpu.sync_copy(tok_hbm.at[pl.ds(base, b_per_w)], idx_v)
        copies = []
        for j in range(n_chunks):
            copies.append(pltpu.async_copy(
                cb_hbm.at[idx_v.at[pl.ds(j * chunk, chunk)]],
                rows_v.at[pl.ds(j * chunk, chunk)],
                sem,
            ))
        for c in copies:
            c.wait()
        pltpu.sync_copy(rows_v, out_hbm.at[pl.ds(base, b_per_w)])

    return k(codebook, tokens)


def kernel(x, W_in, b_in, W1, b1, g1, be1, W2, b2, g2, be2, W_out, b_out,
           codebook):
    x2d = x.reshape(ROWS, IN_DIM)
    row = lambda v: v.reshape(1, -1)
    csq = jnp.sum(codebook * codebook, axis=1)[:, None]
    cbm2 = (-2.0 * codebook).astype(jnp.bfloat16)
    srows = ROWS // NSLICE
    toks, zqs, zs = [], [], []
    for s in range(NSLICE):
        z_s, tok_s = _encode_and_tokenize(
            srows, x2d[s * srows:(s + 1) * srows], W_in, row(b_in),
            W1, row(b1), row(g1), row(be1), W2, row(b2), row(g2), row(be2),
            W_out, row(b_out), cbm2, csq)
        zq_s = _sc_gather(srows, codebook, tok_s)
        toks.append(tok_s)
        zqs.append(zq_s)
        zs.append(z_s)
    tokens = jnp.concatenate(toks)
    z_q = jnp.concatenate(zqs)
    z = jnp.concatenate(zs)
    return (tokens.reshape(B, N),
            z_q.reshape(B, N, CODE_DIM),
            z.reshape(B, N, CODE_DIM))


# speedup vs baseline: 1.2967x; 1.2967x over previous
import functools

import jax
import jax.numpy as jnp
from jax import lax
from jax.experimental import pallas as pl
from jax.experimental.pallas import tpu as pltpu
from jax.experimental.pallas import tpu_sc as plsc

B, N, IN_DIM, HID, CODE_DIM, N_CODES = 16, 4096, 16, 64, 32, 1024
ROWS = B * N
BLK = 2048
NBLK = ROWS // BLK
CCHUNK = 128
NCHUNK = N_CODES // CCHUNK

_EPS = 1e-5
_BIG = 3.0e38


def _bf16_dot(a, b):
    return jnp.dot(a.astype(jnp.bfloat16), b.astype(jnp.bfloat16),
                   preferred_element_type=jnp.float32)


def _tc_body(x_ref, w_in_ref, b_in_ref, w1_ref, b1_ref, g1_ref, be1_ref,
             w2_ref, b2_ref, g2_ref, be2_ref, w_out_ref, b_out_ref, cbm2_ref,
             csq_ref, z_ref, tok_ref):
    h = _bf16_dot(x_ref[...], w_in_ref[...]) + b_in_ref[...]
    for w, b, g, be in ((w1_ref, b1_ref, g1_ref, be1_ref),
                        (w2_ref, b2_ref, g2_ref, be2_ref)):
        t = _bf16_dot(h, w[...]) + b[...]
        m = jnp.mean(t, axis=-1, keepdims=True)
        v = jnp.mean((t - m) * (t - m), axis=-1, keepdims=True)
        t = (t - m) / jnp.sqrt(v + _EPS) * g[...] + be[...]
        h = jnp.maximum(t, 0.0) + h
    z = _bf16_dot(h, w_out_ref[...]) + b_out_ref[...]
    z_ref[...] = z

    zb = z.astype(jnp.bfloat16)

    ids = lax.broadcasted_iota(jnp.int32, (CCHUNK, BLK), 0)
    gmin = jnp.full((BLK,), _BIG, jnp.float32)
    gidx = jnp.zeros((BLK,), jnp.int32)
    gchunk = jnp.zeros((BLK,), jnp.int32)
    for c in range(NCHUNK):
        cbm2 = cbm2_ref[c * CCHUNK:(c + 1) * CCHUNK, :]
        mm = lax.dot_general(cbm2, zb, (((1,), (1,)), ((), ())),
                             preferred_element_type=jnp.float32)
        d = mm + csq_ref[c * CCHUNK:(c + 1) * CCHUNK, 0:1]
        cmin = jnp.min(d, axis=0)
        cidx = jnp.argmin(d, axis=0).astype(jnp.int32)
        better = cmin < gmin
        gmin = jnp.where(better, cmin, gmin)
        gidx = jnp.where(better, cidx, gidx)
        gchunk = jnp.where(better, c, gchunk)
    tok_ref[...] = gchunk * CCHUNK + gidx


def _encode_and_tokenize(x2d, w_in, b_in, w1, b1, g1, be1, w2, b2, g2, be2,
                         w_out, b_out, cbm2, csq, interpret=False):
    full = lambda shape: pl.BlockSpec(shape, lambda i: (0,) * len(shape))
    return pl.pallas_call(
        _tc_body,
        grid=(NBLK,),
        in_specs=[
            pl.BlockSpec((BLK, IN_DIM), lambda i: (i, 0)),
            full((IN_DIM, HID)), full((1, HID)),
            full((HID, HID)), full((1, HID)), full((1, HID)), full((1, HID)),
            full((HID, HID)), full((1, HID)), full((1, HID)), full((1, HID)),
            full((HID, CODE_DIM)), full((1, CODE_DIM)),
            full((N_CODES, CODE_DIM)),
            full((N_CODES, 1)),
        ],
        out_specs=[
            pl.BlockSpec((BLK, CODE_DIM), lambda i: (i, 0)),
            pl.BlockSpec((BLK,), lambda i: (i,)),
        ],
        out_shape=[
            jax.ShapeDtypeStruct((ROWS, CODE_DIM), jnp.float32),
            jax.ShapeDtypeStruct((ROWS,), jnp.int32),
        ],
        interpret=interpret,
    )(x2d, w_in, b_in, w1, b1, g1, be1, w2, b2, g2, be2, w_out, b_out,
      cbm2, csq)


def _sc_gather(codebook, tokens):
    info = plsc.get_sparse_core_info()
    nw = info.num_cores * info.num_subcores
    b_per_w = ROWS // nw
    chunk = 128
    n_chunks = b_per_w // chunk
    mesh = plsc.VectorSubcoreMesh(core_axis_name="c", subcore_axis_name="s")

    @functools.partial(
        pl.kernel, mesh=mesh,
        out_type=jax.ShapeDtypeStruct((ROWS, CODE_DIM), jnp.float32),
        scratch_types=[
            pltpu.VMEM((b_per_w,), jnp.int32),
            pltpu.VMEM((b_per_w, CODE_DIM), jnp.float32),
            pltpu.SemaphoreType.DMA,
        ],
        compiler_params=pltpu.CompilerParams(use_tc_tiling_on_sc=False),
    )
    def k(cb_hbm, tok_hbm, out_hbm, idx_v, rows_v, sem):
        wid = lax.axis_index("s") * info.num_cores + lax.axis_index("c")
        base = wid * b_per_w
        pltpu.sync_copy(tok_hbm.at[pl.ds(base, b_per_w)], idx_v)
        copies = []
        for j in range(n_chunks):
            copies.append(pltpu.async_copy(
                cb_hbm.at[idx_v.at[pl.ds(j * chunk, chunk)]],
                rows_v.at[pl.ds(j * chunk, chunk)],
                sem,
            ))
        for c in copies:
            c.wait()
        pltpu.sync_copy(rows_v, out_hbm.at[pl.ds(base, b_per_w)])

    return k(codebook, tokens)


def kernel(x, W_in, b_in, W1, b1, g1, be1, W2, b2, g2, be2, W_out, b_out,
           codebook):
    x2d = x.reshape(ROWS, IN_DIM)
    row = lambda v: v.reshape(1, -1)
    csq = jnp.sum(codebook * codebook, axis=1)[:, None]
    cbm2 = (-2.0 * codebook).astype(jnp.bfloat16)
    z, tokens = _encode_and_tokenize(
        x2d, W_in, row(b_in), W1, row(b1), row(g1), row(be1),
        W2, row(b2), row(g2), row(be2), W_out, row(b_out), cbm2, csq)
    z_q = _sc_gather(codebook, tokens)
    return (tokens.reshape(B, N),
            z_q.reshape(B, N, CODE_DIM),
            z.reshape(B, N, CODE_DIM))


# speedup vs baseline: 1.2968x; 1.0001x over previous
import functools

import jax
import jax.numpy as jnp
from jax import lax
from jax.experimental import pallas as pl
from jax.experimental.pallas import tpu as pltpu
from jax.experimental.pallas import tpu_sc as plsc

B, N, IN_DIM, HID, CODE_DIM, N_CODES = 16, 4096, 16, 64, 32, 1024
ROWS = B * N
BLK = 2048
NBLK = ROWS // BLK
CCHUNK = 128
NCHUNK = N_CODES // CCHUNK

_EPS = 1e-5
_BIG = 3.0e38


def _bf16_dot(a, b):
    return jnp.dot(a.astype(jnp.bfloat16), b.astype(jnp.bfloat16),
                   preferred_element_type=jnp.float32)


def _tc_body(x_ref, w_in_ref, b_in_ref, w1_ref, b1_ref, g1_ref, be1_ref,
             w2_ref, b2_ref, g2_ref, be2_ref, w_out_ref, b_out_ref, cbm2_ref,
             csq_ref, z_ref, tok_ref):
    h = _bf16_dot(x_ref[...], w_in_ref[...]) + b_in_ref[...]
    for w, b, g, be in ((w1_ref, b1_ref, g1_ref, be1_ref),
                        (w2_ref, b2_ref, g2_ref, be2_ref)):
        t = _bf16_dot(h, w[...]) + b[...]
        m = jnp.mean(t, axis=-1, keepdims=True)
        v = jnp.mean((t - m) * (t - m), axis=-1, keepdims=True)
        t = (t - m) / jnp.sqrt(v + _EPS) * g[...] + be[...]
        h = jnp.maximum(t, 0.0) + h
    z = _bf16_dot(h, w_out_ref[...]) + b_out_ref[...]
    z_ref[...] = z

    zb = z.astype(jnp.bfloat16)

    ids = lax.broadcasted_iota(jnp.int32, (CCHUNK, BLK), 0)
    gmin = jnp.full((BLK,), _BIG, jnp.float32)
    gidx = jnp.zeros((BLK,), jnp.int32)
    gchunk = jnp.zeros((BLK,), jnp.int32)
    for c in range(NCHUNK):
        cbm2 = cbm2_ref[c * CCHUNK:(c + 1) * CCHUNK, :]
        mm = lax.dot_general(cbm2, zb, (((1,), (1,)), ((), ())),
                             preferred_element_type=jnp.float32)
        d = mm + csq_ref[c * CCHUNK:(c + 1) * CCHUNK, 0:1]
        cmin = jnp.min(d, axis=0)
        cidx = jnp.argmin(d, axis=0).astype(jnp.int32)
        better = cmin < gmin
        gmin = jnp.where(better, cmin, gmin)
        gidx = jnp.where(better, cidx, gidx)
        gchunk = jnp.where(better, c, gchunk)
    tok_ref[...] = gchunk * CCHUNK + gidx


def _encode_and_tokenize(x2d, w_in, b_in, w1, b1, g1, be1, w2, b2, g2, be2,
                         w_out, b_out, cbm2, csq, interpret=False):
    full = lambda shape: pl.BlockSpec(shape, lambda i: (0,) * len(shape))
    return pl.pallas_call(
        _tc_body,
        grid=(NBLK,),
        in_specs=[
            pl.BlockSpec((BLK, IN_DIM), lambda i: (i, 0)),
            full((IN_DIM, HID)), full((1, HID)),
            full((HID, HID)), full((1, HID)), full((1, HID)), full((1, HID)),
            full((HID, HID)), full((1, HID)), full((1, HID)), full((1, HID)),
            full((HID, CODE_DIM)), full((1, CODE_DIM)),
            full((N_CODES, CODE_DIM)),
            full((N_CODES, 1)),
        ],
        out_specs=[
            pl.BlockSpec((BLK, CODE_DIM), lambda i: (i, 0)),
            pl.BlockSpec((BLK,), lambda i: (i,)),
        ],
        out_shape=[
            jax.ShapeDtypeStruct((ROWS, CODE_DIM), jnp.float32),
            jax.ShapeDtypeStruct((ROWS,), jnp.int32),
        ],
        interpret=interpret,
    )(x2d, w_in, b_in, w1, b1, g1, be1, w2, b2, g2, be2, w_out, b_out,
      cbm2, csq)


def _sc_gather(codebook, tokens):
    info = plsc.get_sparse_core_info()
    nw = info.num_cores * info.num_subcores
    b_per_w = ROWS // nw
    chunk = 128
    n_chunks = b_per_w // chunk
    mesh = plsc.VectorSubcoreMesh(core_axis_name="c", subcore_axis_name="s")

    @functools.partial(
        pl.kernel, mesh=mesh,
        out_type=jax.ShapeDtypeStruct((ROWS, CODE_DIM), jnp.float32),
        scratch_types=[
            pltpu.VMEM((b_per_w,), jnp.int32),
            pltpu.VMEM((b_per_w, CODE_DIM), jnp.float32),
            pltpu.SemaphoreType.DMA,
        ],
        compiler_params=pltpu.CompilerParams(use_tc_tiling_on_sc=False,
                                             needs_layout_passes=False),
    )
    def k(cb_hbm, tok_hbm, out_hbm, idx_v, rows_v, sem):
        wid = lax.axis_index("s") * info.num_cores + lax.axis_index("c")
        base = wid * b_per_w
        pltpu.sync_copy(tok_hbm.at[pl.ds(base, b_per_w)], idx_v)
        copies = []
        for j in range(n_chunks):
            copies.append(pltpu.async_copy(
                cb_hbm.at[idx_v.at[pl.ds(j * chunk, chunk)]],
                rows_v.at[pl.ds(j * chunk, chunk)],
                sem,
            ))
        for c in copies:
            c.wait()
        pltpu.sync_copy(rows_v, out_hbm.at[pl.ds(base, b_per_w)])

    return k(codebook, tokens)


def kernel(x, W_in, b_in, W1, b1, g1, be1, W2, b2, g2, be2, W_out, b_out,
           codebook):
    x2d = x.reshape(ROWS, IN_DIM)
    row = lambda v: v.reshape(1, -1)
    csq = jnp.sum(codebook * codebook, axis=1)[:, None]
    cbm2 = (-2.0 * codebook).astype(jnp.bfloat16)
    z, tokens = _encode_and_tokenize(
        x2d, W_in, row(b_in), W1, row(b1), row(g1), row(be1),
        W2, row(b2), row(g2), row(be2), W_out, row(b_out), cbm2, csq)
    z_q = _sc_gather(codebook, tokens)
    return (tokens.reshape(B, N),
            z_q.reshape(B, N, CODE_DIM),
            z.reshape(B, N, CODE_DIM))


# speedup vs baseline: 1.4061x; 1.0843x over previous
import functools

import jax
import jax.numpy as jnp
from jax import lax
from jax.experimental import pallas as pl
from jax.experimental.pallas import tpu as pltpu
from jax.experimental.pallas import tpu_sc as plsc

B, N, IN_DIM, HID, CODE_DIM, N_CODES = 16, 4096, 16, 64, 32, 1024
ROWS = B * N
BLK = 4096
NBLK = ROWS // BLK
CCHUNK = 512
NCHUNK = N_CODES // CCHUNK

_EPS = 1e-5
_BIG = 3.0e38


def _bf16_dot(a, b):
    return jnp.dot(a.astype(jnp.bfloat16), b.astype(jnp.bfloat16),
                   preferred_element_type=jnp.float32)


def _tc_body(x_ref, w_in_ref, b_in_ref, w1_ref, b1_ref, g1_ref, be1_ref,
             w2_ref, b2_ref, g2_ref, be2_ref, w_out_ref, b_out_ref, cbm2_ref,
             csq_ref, z_ref, tok_ref):
    h = _bf16_dot(x_ref[...], w_in_ref[...]) + b_in_ref[...]
    for w, b, g, be in ((w1_ref, b1_ref, g1_ref, be1_ref),
                        (w2_ref, b2_ref, g2_ref, be2_ref)):
        t = _bf16_dot(h, w[...]) + b[...]
        m = jnp.mean(t, axis=-1, keepdims=True)
        v = jnp.mean((t - m) * (t - m), axis=-1, keepdims=True)
        t = (t - m) / jnp.sqrt(v + _EPS) * g[...] + be[...]
        h = jnp.maximum(t, 0.0) + h
    z = _bf16_dot(h, w_out_ref[...]) + b_out_ref[...]
    z_ref[...] = z

    zb = z.astype(jnp.bfloat16)

    ids = lax.broadcasted_iota(jnp.int32, (CCHUNK, BLK), 0)
    gmin = jnp.full((BLK,), _BIG, jnp.float32)
    gidx = jnp.zeros((BLK,), jnp.int32)
    gchunk = jnp.zeros((BLK,), jnp.int32)
    for c in range(NCHUNK):
        cbm2 = cbm2_ref[c * CCHUNK:(c + 1) * CCHUNK, :]
        mm = lax.dot_general(cbm2, zb, (((1,), (1,)), ((), ())),
                             preferred_element_type=jnp.float32)
        d = mm + csq_ref[c * CCHUNK:(c + 1) * CCHUNK, 0:1]
        cmin = jnp.min(d, axis=0)
        cidx = jnp.argmin(d, axis=0).astype(jnp.int32)
        better = cmin < gmin
        gmin = jnp.where(better, cmin, gmin)
        gidx = jnp.where(better, cidx, gidx)
        gchunk = jnp.where(better, c, gchunk)
    tok_ref[...] = gchunk * CCHUNK + gidx


def _encode_and_tokenize(x2d, w_in, b_in, w1, b1, g1, be1, w2, b2, g2, be2,
                         w_out, b_out, cbm2, csq, interpret=False):
    full = lambda shape: pl.BlockSpec(shape, lambda i: (0,) * len(shape))
    return pl.pallas_call(
        _tc_body,
        grid=(NBLK,),
        in_specs=[
            pl.BlockSpec((BLK, IN_DIM), lambda i: (i, 0)),
            full((IN_DIM, HID)), full((1, HID)),
            full((HID, HID)), full((1, HID)), full((1, HID)), full((1, HID)),
            full((HID, HID)), full((1, HID)), full((1, HID)), full((1, HID)),
            full((HID, CODE_DIM)), full((1, CODE_DIM)),
            full((N_CODES, CODE_DIM)),
            full((N_CODES, 1)),
        ],
        out_specs=[
            pl.BlockSpec((BLK, CODE_DIM), lambda i: (i, 0)),
            pl.BlockSpec((BLK,), lambda i: (i,)),
        ],
        out_shape=[
            jax.ShapeDtypeStruct((ROWS, CODE_DIM), jnp.float32),
            jax.ShapeDtypeStruct((ROWS,), jnp.int32),
        ],
        interpret=interpret,
    )(x2d, w_in, b_in, w1, b1, g1, be1, w2, b2, g2, be2, w_out, b_out,
      cbm2, csq)


def _sc_gather(codebook, tokens):
    info = plsc.get_sparse_core_info()
    nw = info.num_cores * info.num_subcores
    b_per_w = ROWS // nw
    chunk = 128
    n_chunks = b_per_w // chunk
    mesh = plsc.VectorSubcoreMesh(core_axis_name="c", subcore_axis_name="s")

    @functools.partial(
        pl.kernel, mesh=mesh,
        out_type=jax.ShapeDtypeStruct((ROWS, CODE_DIM), jnp.float32),
        scratch_types=[
            pltpu.VMEM((b_per_w,), jnp.int32),
            pltpu.VMEM((b_per_w, CODE_DIM), jnp.float32),
            pltpu.SemaphoreType.DMA,
        ],
        compiler_params=pltpu.CompilerParams(use_tc_tiling_on_sc=False),
    )
    def k(cb_hbm, tok_hbm, out_hbm, idx_v, rows_v, sem):
        wid = lax.axis_index("s") * info.num_cores + lax.axis_index("c")
        base = wid * b_per_w
        pltpu.sync_copy(tok_hbm.at[pl.ds(base, b_per_w)], idx_v)
        copies = []
        for j in range(n_chunks):
            copies.append(pltpu.async_copy(
                cb_hbm.at[idx_v.at[pl.ds(j * chunk, chunk)]],
                rows_v.at[pl.ds(j * chunk, chunk)],
                sem,
            ))
        for c in copies:
            c.wait()
        pltpu.sync_copy(rows_v, out_hbm.at[pl.ds(base, b_per_w)])

    return k(codebook, tokens)


def kernel(x, W_in, b_in, W1, b1, g1, be1, W2, b2, g2, be2, W_out, b_out,
           codebook):
    x2d = x.reshape(ROWS, IN_DIM)
    row = lambda v: v.reshape(1, -1)
    csq = jnp.sum(codebook * codebook, axis=1)[:, None]
    cbm2 = (-2.0 * codebook).astype(jnp.bfloat16)
    z, tokens = _encode_and_tokenize(
        x2d, W_in, row(b_in), W1, row(b1), row(g1), row(be1),
        W2, row(b2), row(g2), row(be2), W_out, row(b_out), cbm2, csq)
    z_q = _sc_gather(codebook, tokens)
    return (tokens.reshape(B, N),
            z_q.reshape(B, N, CODE_DIM),
            z.reshape(B, N, CODE_DIM))


# speedup vs baseline: 1.4340x; 1.0198x over previous
import functools

import jax
import jax.numpy as jnp
from jax import lax
from jax.experimental import pallas as pl
from jax.experimental.pallas import tpu as pltpu
from jax.experimental.pallas import tpu_sc as plsc

B, N, IN_DIM, HID, CODE_DIM, N_CODES = 16, 4096, 16, 64, 32, 1024
ROWS = B * N
BLK = 8192
NBLK = ROWS // BLK
CCHUNK = 1024
NCHUNK = N_CODES // CCHUNK

_EPS = 1e-5
_BIG = 3.0e38


def _bf16_dot(a, b):
    return jnp.dot(a.astype(jnp.bfloat16), b.astype(jnp.bfloat16),
                   preferred_element_type=jnp.float32)


def _tc_body(x_ref, w_in_ref, b_in_ref, w1_ref, b1_ref, g1_ref, be1_ref,
             w2_ref, b2_ref, g2_ref, be2_ref, w_out_ref, b_out_ref, cbm2_ref,
             csq_ref, z_ref, tok_ref):
    h = _bf16_dot(x_ref[...], w_in_ref[...]) + b_in_ref[...]
    for w, b, g, be in ((w1_ref, b1_ref, g1_ref, be1_ref),
                        (w2_ref, b2_ref, g2_ref, be2_ref)):
        t = _bf16_dot(h, w[...]) + b[...]
        m = jnp.mean(t, axis=-1, keepdims=True)
        v = jnp.mean((t - m) * (t - m), axis=-1, keepdims=True)
        t = (t - m) / jnp.sqrt(v + _EPS) * g[...] + be[...]
        h = jnp.maximum(t, 0.0) + h
    z = _bf16_dot(h, w_out_ref[...]) + b_out_ref[...]
    z_ref[...] = z

    zb = z.astype(jnp.bfloat16)

    ids = lax.broadcasted_iota(jnp.int32, (CCHUNK, BLK), 0)
    gmin = jnp.full((BLK,), _BIG, jnp.float32)
    gidx = jnp.zeros((BLK,), jnp.int32)
    gchunk = jnp.zeros((BLK,), jnp.int32)
    for c in range(NCHUNK):
        cbm2 = cbm2_ref[c * CCHUNK:(c + 1) * CCHUNK, :]
        mm = lax.dot_general(cbm2, zb, (((1,), (1,)), ((), ())),
                             preferred_element_type=jnp.float32)
        d = mm + csq_ref[c * CCHUNK:(c + 1) * CCHUNK, 0:1]
        cmin = jnp.min(d, axis=0)
        cidx = jnp.argmin(d, axis=0).astype(jnp.int32)
        better = cmin < gmin
        gmin = jnp.where(better, cmin, gmin)
        gidx = jnp.where(better, cidx, gidx)
        gchunk = jnp.where(better, c, gchunk)
    tok_ref[...] = gchunk * CCHUNK + gidx


def _encode_and_tokenize(x2d, w_in, b_in, w1, b1, g1, be1, w2, b2, g2, be2,
                         w_out, b_out, cbm2, csq, interpret=False):
    full = lambda shape: pl.BlockSpec(shape, lambda i: (0,) * len(shape))
    return pl.pallas_call(
        _tc_body,
        grid=(NBLK,),
        in_specs=[
            pl.BlockSpec((BLK, IN_DIM), lambda i: (i, 0)),
            full((IN_DIM, HID)), full((1, HID)),
            full((HID, HID)), full((1, HID)), full((1, HID)), full((1, HID)),
            full((HID, HID)), full((1, HID)), full((1, HID)), full((1, HID)),
            full((HID, CODE_DIM)), full((1, CODE_DIM)),
            full((N_CODES, CODE_DIM)),
            full((N_CODES, 1)),
        ],
        out_specs=[
            pl.BlockSpec((BLK, CODE_DIM), lambda i: (i, 0)),
            pl.BlockSpec((BLK,), lambda i: (i,)),
        ],
        out_shape=[
            jax.ShapeDtypeStruct((ROWS, CODE_DIM), jnp.float32),
            jax.ShapeDtypeStruct((ROWS,), jnp.int32),
        ],
        interpret=interpret,
    )(x2d, w_in, b_in, w1, b1, g1, be1, w2, b2, g2, be2, w_out, b_out,
      cbm2, csq)


def _sc_gather(codebook, tokens):
    info = plsc.get_sparse_core_info()
    nw = info.num_cores * info.num_subcores
    b_per_w = ROWS // nw
    chunk = 128
    n_chunks = b_per_w // chunk
    mesh = plsc.VectorSubcoreMesh(core_axis_name="c", subcore_axis_name="s")

    @functools.partial(
        pl.kernel, mesh=mesh,
        out_type=jax.ShapeDtypeStruct((ROWS, CODE_DIM), jnp.float32),
        scratch_types=[
            pltpu.VMEM((b_per_w,), jnp.int32),
            pltpu.VMEM((b_per_w, CODE_DIM), jnp.float32),
            pltpu.SemaphoreType.DMA,
        ],
        compiler_params=pltpu.CompilerParams(use_tc_tiling_on_sc=False),
    )
    def k(cb_hbm, tok_hbm, out_hbm, idx_v, rows_v, sem):
        wid = lax.axis_index("s") * info.num_cores + lax.axis_index("c")
        base = wid * b_per_w
        pltpu.sync_copy(tok_hbm.at[pl.ds(base, b_per_w)], idx_v)
        copies = []
        for j in range(n_chunks):
            copies.append(pltpu.async_copy(
                cb_hbm.at[idx_v.at[pl.ds(j * chunk, chunk)]],
                rows_v.at[pl.ds(j * chunk, chunk)],
                sem,
            ))
        for c in copies:
            c.wait()
        pltpu.sync_copy(rows_v, out_hbm.at[pl.ds(base, b_per_w)])

    return k(codebook, tokens)


def kernel(x, W_in, b_in, W1, b1, g1, be1, W2, b2, g2, be2, W_out, b_out,
           codebook):
    x2d = x.reshape(ROWS, IN_DIM)
    row = lambda v: v.reshape(1, -1)
    csq = jnp.sum(codebook * codebook, axis=1)[:, None]
    cbm2 = (-2.0 * codebook).astype(jnp.bfloat16)
    z, tokens = _encode_and_tokenize(
        x2d, W_in, row(b_in), W1, row(b1), row(g1), row(be1),
        W2, row(b2), row(g2), row(be2), W_out, row(b_out), cbm2, csq)
    z_q = _sc_gather(codebook, tokens)
    return (tokens.reshape(B, N),
            z_q.reshape(B, N, CODE_DIM),
            z.reshape(B, N, CODE_DIM))
